# baseline (speedup 1.0000x reference)
import functools

import jax
import jax.numpy as jnp
from jax import lax
from jax.experimental import pallas as pl
from jax.experimental.pallas import tpu as pltpu
from jax.experimental.pallas import tpu_sc as plsc

_LF = 0.15875
_VGOAL = 0.9
_N = 100000
_COLS = 6
_NC = 2
_NS = 16
_NW = _NC * _NS
_L = 16
_RPW = _N // _NW
_ITERS = (_RPW + _L - 1) // _L
_ROWS_COPY = _RPW + 3
_ELEMS_COPY = _ROWS_COPY * _COLS

_BIG = 3.4e38


def _sc_search_body(flat_hbm, pvec_hbm, out_hbm, wp_v, pv_v, outs_v):
  wid = lax.axis_index("s") * _NC + lax.axis_index("c")
  row0 = wid * _RPW
  base_row = (row0 // 4) * 4
  rem = row0 - base_row
  start_elem = pl.multiple_of(base_row * _COLS, 8)
  pltpu.sync_copy(flat_hbm.at[pl.ds(start_elem, _ELEMS_COPY)], wp_v)
  pltpu.sync_copy(pvec_hbm, pv_v)

  lanes = lax.broadcasted_iota(jnp.int32, (_L,), 0)
  zeros_i = jnp.zeros((_L,), jnp.int32)
  pcx = plsc.load_gather(pv_v, [zeros_i])
  pcy = plsc.load_gather(pv_v, [zeros_i + 1])

  def body(i, carry):
    bd, br = carry
    r = i * _L + lanes
    rr = jnp.minimum(r, _RPW - 1)
    base = (rem + rr) * _COLS
    xv = plsc.load_gather(wp_v, [base + 1])
    yv = plsc.load_gather(wp_v, [base + 2])
    dx = xv - pcx
    dy = yv - pcy
    d2 = dx * dx + dy * dy
    upd = d2 < bd
    return jnp.where(upd, d2, bd), jnp.where(upd, rr, br)

  bd0 = jnp.full((_L,), _BIG, jnp.float32)
  bd, br = lax.fori_loop(0, _ITERS, body, (bd0, zeros_i))

  m = jnp.min(bd)
  tied = bd == m
  r_win = jnp.min(jnp.where(tied, br, jnp.int32(0x7FFFFFFF)))
  idx_row = (rem + r_win) * _COLS + jnp.clip(lanes - 2, 0, _COLS - 1)
  rowdata = plsc.load_gather(wp_v, [idx_row])
  g_row = (row0 + r_win).astype(jnp.float32)
  outvec = jnp.where(lanes == 0, m,
                     jnp.where(lanes == 1, g_row,
                               jnp.where(lanes >= 2, rowdata, rowdata)))
  outs_v[...] = outvec
  pltpu.sync_copy(outs_v, out_hbm.at[wid])


@functools.cache
def _get_sc_search():
  return pl.kernel(
      _sc_search_body,
      out_type=jax.ShapeDtypeStruct((_NW, _L), jnp.float32),
      mesh=plsc.VectorSubcoreMesh(core_axis_name="c", subcore_axis_name="s",
                                  num_cores=_NC, num_subcores=_NS),
      compiler_params=pltpu.CompilerParams(needs_layout_passes=False),
      scratch_types=[
          pltpu.VMEM((_ELEMS_COPY,), jnp.float32),
          pltpu.VMEM((_L,), jnp.float32),
          pltpu.VMEM((_L,), jnp.float32),
      ],
  )


def _atan(u):
  t = jnp.abs(u)
  inv = t > 1.0
  z = jnp.where(inv, 1.0 / jnp.maximum(t, 1e-30), t)
  big = z > 0.4142135623730951
  z2 = jnp.where(big, (z - 1.0) / (z + 1.0), z)
  w = z2 * z2
  p = ((8.05374449538e-2 * w - 1.38776856032e-1) * w
       + 1.99777106478e-1) * w - 3.33329491539e-1
  r = z2 + z2 * w * p
  r = jnp.where(big, jnp.float32(0.7853981633974483) + r, r)
  r = jnp.where(inv, jnp.float32(1.5707963267948966) - r, r)
  return jnp.where(u < 0.0, -r, r)


def _merge_body(pv_ref, cand_ref, out_ref):
  cand = cand_ref[...]
  lane = lax.broadcasted_iota(jnp.int32, (_NW, _L), 1)
  d2f = jnp.where(lane == 0, cand, _BIG)
  m = jnp.min(d2f)
  rowd2 = jnp.min(d2f, axis=1, keepdims=True)
  rowidx = jnp.max(jnp.where(lane == 1, cand, -_BIG), axis=1, keepdims=True)
  idxsel = jnp.where(rowd2 <= m, rowidx, _BIG)
  istar = jnp.min(idxsel)
  sel = idxsel <= istar
  row = jnp.sum(jnp.where(sel, cand, 0.0), axis=0, keepdims=True)
  lane1 = lax.broadcasted_iota(jnp.int32, (1, _L), 1)
  wx = jnp.sum(jnp.where(lane1 == 3, row, 0.0))
  wy = jnp.sum(jnp.where(lane1 == 4, row, 0.0))
  wh = jnp.sum(jnp.where(lane1 == 5, row, 0.0))
  ws = jnp.sum(jnp.where(lane1 == 7, row, 0.0))

  pcx = pv_ref[0]
  pcy = pv_ref[1]
  fav0 = pv_ref[2]
  fav1 = pv_ref[3]
  thetap = pv_ref[4]
  k_e = pv_ref[5]
  k_h = pv_ref[6]

  ce = (pcx - wx) * fav0 + (pcy - wy) * fav1
  pi = jnp.float32(jnp.pi)
  he = jnp.remainder(wh - thetap + pi, 2.0 * pi) - pi
  v = ws * jnp.float32(_VGOAL)
  steer = k_h * he + _atan(k_e * -ce / (v + 1e-05))
  ol = lax.broadcasted_iota(jnp.int32, (1, 8), 1)
  res = jnp.where(ol == 0, steer,
                  jnp.where(ol == 1, v,
                            jnp.where(ol == 2, ce, he)))
  out_ref[...] = res


_merge = pl.pallas_call(
    _merge_body,
    out_shape=jax.ShapeDtypeStruct((1, 8), jnp.float32),
    in_specs=[
        pl.BlockSpec(memory_space=pltpu.SMEM),
        pl.BlockSpec(memory_space=pltpu.VMEM),
    ],
    out_specs=pl.BlockSpec(memory_space=pltpu.VMEM),
)


def kernel(pose, waypoints, k_e, k_h):
  pcx = pose[0] + _LF * jnp.sin(pose[2])
  pcy = pose[1] + _LF * jnp.cos(pose[2])
  thetap = jnp.remainder(pose[2] + jnp.pi, 2.0 * jnp.pi)
  fav0 = -jnp.cos(thetap + jnp.pi / 2.0)
  fav1 = -jnp.sin(thetap + jnp.pi / 2.0)
  pvec_sc = jnp.zeros((_L,), jnp.float32).at[0].set(pcx).at[1].set(pcy)
  flat = waypoints.reshape(-1)
  cand = _get_sc_search()(flat, pvec_sc)
  pvec_tc = jnp.stack([pcx, pcy, fav0, fav1, thetap,
                       k_e.astype(jnp.float32), k_h.astype(jnp.float32),
                       jnp.float32(0.0)])
  out = _merge(pvec_tc, cand)
  return (out[0, 0], out[0, 1], out[0, 2], out[0, 3])


# speedup vs baseline: 2.3627x; 2.3627x over previous
import functools

import jax
import jax.numpy as jnp
from jax import lax
from jax.experimental import pallas as pl
from jax.experimental.pallas import tpu as pltpu
from jax.experimental.pallas import tpu_sc as plsc

_LF = 0.15875
_VGOAL = 0.9
_N = 100000
_NC = 2
_NS = 16
_NW = _NC * _NS
_L = 16
_RPW = _N // _NW
_ITERS = (_RPW + _L - 1) // _L
_WIN = 3152
_PADN = 100024

_BIG = 3.4e38


def _sc_search_body(wx_hbm, wy_hbm, wh_hbm, ws_hbm, pvec_hbm, out_hbm,
                    x_v, y_v, h_v, s_v, pv_v, outs_v, semx, semy, semh, sems):
  wid = lax.axis_index("s") * _NC + lax.axis_index("c")
  row0 = wid * _RPW
  base = pl.multiple_of((row0 // 8) * 8, 8)
  rem = row0 - base
  pltpu.sync_copy(pvec_hbm, pv_v)
  cx = pltpu.async_copy(wx_hbm.at[pl.ds(base, _WIN)], x_v, semx)
  cy = pltpu.async_copy(wy_hbm.at[pl.ds(base, _WIN)], y_v, semy)
  ch = pltpu.async_copy(wh_hbm.at[pl.ds(base, _WIN)], h_v, semh)
  cs = pltpu.async_copy(ws_hbm.at[pl.ds(base, _WIN)], s_v, sems)
  cx.wait()
  cy.wait()

  lanes = lax.broadcasted_iota(jnp.int32, (_L,), 0)
  zeros_i = jnp.zeros((_L,), jnp.int32)
  pv16 = pv_v[...]
  pcx = jnp.sum(jnp.where(lanes == 0, pv16, 0.0))
  pcy = jnp.sum(jnp.where(lanes == 1, pv16, 0.0))

  def body(i, carry):
    bd, br = carry
    l = i * _L + lanes
    xv = x_v[pl.ds(i * _L, _L)]
    yv = y_v[pl.ds(i * _L, _L)]
    dx = xv - pcx
    dy = yv - pcy
    d2 = dx * dx + dy * dy
    ok = (l >= rem) & (l < rem + _RPW)
    d2 = jnp.where(ok, d2, _BIG)
    upd = d2 < bd
    return jnp.where(upd, d2, bd), jnp.where(upd, l, br)

  bd0 = jnp.full((_L,), _BIG, jnp.float32)
  bd, br = lax.fori_loop(0, _WIN // _L, body, (bd0, zeros_i))

  m = jnp.min(bd)
  r_win = jnp.min(jnp.where(bd == m, br, jnp.int32(0x7FFFFFFF)))
  ch.wait()
  cs.wait()
  r_al = pl.multiple_of((r_win // _L) * _L, _L)
  jsel = lanes == (r_win - r_al)
  xw = jnp.sum(jnp.where(jsel, x_v[pl.ds(r_al, _L)], 0.0))
  yw = jnp.sum(jnp.where(jsel, y_v[pl.ds(r_al, _L)], 0.0))
  hw = jnp.sum(jnp.where(jsel, h_v[pl.ds(r_al, _L)], 0.0))
  sw = jnp.sum(jnp.where(jsel, s_v[pl.ds(r_al, _L)], 0.0))
  g_row = (base + r_win).astype(jnp.float32)
  outvec = jnp.where(lanes == 0, m,
                     jnp.where(lanes == 1, g_row,
                               jnp.where(lanes == 3, xw,
                                         jnp.where(lanes == 4, yw,
                                                   jnp.where(lanes == 5, hw,
                                                             sw)))))
  outs_v[...] = outvec
  pltpu.sync_copy(outs_v, out_hbm.at[wid])


@functools.cache
def _get_sc_search():
  return pl.kernel(
      _sc_search_body,
      out_type=jax.ShapeDtypeStruct((_NW, _L), jnp.float32),
      mesh=plsc.VectorSubcoreMesh(core_axis_name="c", subcore_axis_name="s",
                                  num_cores=_NC, num_subcores=_NS),
      compiler_params=pltpu.CompilerParams(needs_layout_passes=False),
      scratch_types=[
          pltpu.VMEM((_WIN,), jnp.float32),
          pltpu.VMEM((_WIN,), jnp.float32),
          pltpu.VMEM((_WIN,), jnp.float32),
          pltpu.VMEM((_WIN,), jnp.float32),
          pltpu.VMEM((_L,), jnp.float32),
          pltpu.VMEM((_L,), jnp.float32),
          pltpu.SemaphoreType.DMA,
          pltpu.SemaphoreType.DMA,
          pltpu.SemaphoreType.DMA,
          pltpu.SemaphoreType.DMA,
      ],
  )


def _atan(u):
  t = jnp.abs(u)
  inv = t > 1.0
  z = jnp.where(inv, 1.0 / jnp.maximum(t, 1e-30), t)
  big = z > 0.4142135623730951
  z2 = jnp.where(big, (z - 1.0) / (z + 1.0), z)
  w = z2 * z2
  p = ((8.05374449538e-2 * w - 1.38776856032e-1) * w
       + 1.99777106478e-1) * w - 3.33329491539e-1
  r = z2 + z2 * w * p
  r = jnp.where(big, jnp.float32(0.7853981633974483) + r, r)
  r = jnp.where(inv, jnp.float32(1.5707963267948966) - r, r)
  return jnp.where(u < 0.0, -r, r)


def _merge_body(pv_ref, cand_ref, out_ref):
  cand = cand_ref[...]
  lane = lax.broadcasted_iota(jnp.int32, (_NW, _L), 1)
  d2f = jnp.where(lane == 0, cand, _BIG)
  m = jnp.min(d2f)
  rowd2 = jnp.min(d2f, axis=1, keepdims=True)
  rowidx = jnp.max(jnp.where(lane == 1, cand, -_BIG), axis=1, keepdims=True)
  idxsel = jnp.where(rowd2 <= m, rowidx, _BIG)
  istar = jnp.min(idxsel)
  sel = idxsel <= istar
  row = jnp.sum(jnp.where(sel, cand, 0.0), axis=0, keepdims=True)
  lane1 = lax.broadcasted_iota(jnp.int32, (1, _L), 1)
  wx = jnp.sum(jnp.where(lane1 == 3, row, 0.0))
  wy = jnp.sum(jnp.where(lane1 == 4, row, 0.0))
  wh = jnp.sum(jnp.where(lane1 == 5, row, 0.0))
  ws = jnp.sum(jnp.where(lane1 == 7, row, 0.0))

  pcx = pv_ref[0]
  pcy = pv_ref[1]
  fav0 = pv_ref[2]
  fav1 = pv_ref[3]
  thetap = pv_ref[4]
  k_e = pv_ref[5]
  k_h = pv_ref[6]

  ce = (pcx - wx) * fav0 + (pcy - wy) * fav1
  pi = jnp.float32(jnp.pi)
  he = jnp.remainder(wh - thetap + pi, 2.0 * pi) - pi
  v = ws * jnp.float32(_VGOAL)
  steer = k_h * he + _atan(k_e * -ce / (v + 1e-05))
  ol = lax.broadcasted_iota(jnp.int32, (1, 8), 1)
  res = jnp.where(ol == 0, steer,
                  jnp.where(ol == 1, v,
                            jnp.where(ol == 2, ce, he)))
  out_ref[...] = res


_merge = pl.pallas_call(
    _merge_body,
    out_shape=jax.ShapeDtypeStruct((1, 8), jnp.float32),
    in_specs=[
        pl.BlockSpec(memory_space=pltpu.SMEM),
        pl.BlockSpec(memory_space=pltpu.VMEM),
    ],
    out_specs=pl.BlockSpec(memory_space=pltpu.VMEM),
)


def kernel(pose, waypoints, k_e, k_h):
  pcx = pose[0] + _LF * jnp.sin(pose[2])
  pcy = pose[1] + _LF * jnp.cos(pose[2])
  thetap = jnp.remainder(pose[2] + jnp.pi, 2.0 * jnp.pi)
  fav0 = -jnp.cos(thetap + jnp.pi / 2.0)
  fav1 = -jnp.sin(thetap + jnp.pi / 2.0)
  pvec_sc = jnp.zeros((_L,), jnp.float32).at[0].set(pcx).at[1].set(pcy)
  pad = (0, _PADN - _N)
  wx = jnp.pad(waypoints[:, 1], pad)
  wy = jnp.pad(waypoints[:, 2], pad)
  wh = jnp.pad(waypoints[:, 3], pad)
  ws = jnp.pad(waypoints[:, 5], pad)
  cand = _get_sc_search()(wx, wy, wh, ws, pvec_sc)
  pvec_tc = jnp.stack([pcx, pcy, fav0, fav1, thetap,
                       k_e.astype(jnp.float32), k_h.astype(jnp.float32),
                       jnp.float32(0.0)])
  out = _merge(pvec_tc, cand)
  return (out[0, 0], out[0, 1], out[0, 2], out[0, 3])


# speedup vs baseline: 2.8357x; 1.2002x over previous
import functools

import jax
import jax.numpy as jnp
from jax import lax
from jax.experimental import pallas as pl
from jax.experimental.pallas import tpu as pltpu
from jax.experimental.pallas import tpu_sc as plsc

_LF = 0.15875
_VGOAL = 0.9
_N = 100000
_NC = 2
_NS = 16
_NW = _NC * _NS
_L = 16
_RPW = _N // _NW
_WIN = 3136
_UNROLL = 14

_BIG = 3.4e38


def _sc_search_body(wx_hbm, wy_hbm, pvec_hbm, out_hbm,
                    x_v, y_v, pv_v, outs_v, semx, semy):
  wid = lax.axis_index("s") * _NC + lax.axis_index("c")
  row0 = wid * _RPW
  base = pl.multiple_of(jnp.minimum((row0 // 8) * 8, _N - _WIN), 8)
  pltpu.sync_copy(pvec_hbm, pv_v)
  cx = pltpu.async_copy(wx_hbm.at[pl.ds(base, _WIN)], x_v, semx)
  cy = pltpu.async_copy(wy_hbm.at[pl.ds(base, _WIN)], y_v, semy)
  cx.wait()
  cy.wait()

  lanes = lax.broadcasted_iota(jnp.int32, (_L,), 0)
  zeros_i = jnp.zeros((_L,), jnp.int32)
  pv16 = pv_v[...]
  pcx = jnp.sum(jnp.where(lanes == 0, pv16, 0.0))
  pcy = jnp.sum(jnp.where(lanes == 1, pv16, 0.0))

  bd0 = jnp.full((_L,), _BIG, jnp.float32)

  @plsc.parallel_loop(0, _WIN, _L, unroll=_UNROLL, carry=(bd0, zeros_i))
  def _loop(i, carry):
    bd, br = carry
    xv = x_v[pl.ds(i, _L)]
    yv = y_v[pl.ds(i, _L)]
    dx = xv - pcx
    dy = yv - pcy
    d2 = dx * dx + dy * dy
    upd = d2 < bd
    return jnp.where(upd, d2, bd), jnp.where(upd, i + lanes, br)

  bd, br = _loop
  m = jnp.min(bd)
  r_win = jnp.min(jnp.where(bd == m, br, jnp.int32(0x7FFFFFFF)))
  g_row = (base + r_win).astype(jnp.float32)
  outvec = jnp.where(lanes == 0, m,
                     jnp.where(lanes == 1, g_row, 0.0))
  outs_v[...] = outvec
  pltpu.sync_copy(outs_v, out_hbm.at[wid])


@functools.cache
def _get_sc_search():
  return pl.kernel(
      _sc_search_body,
      out_type=jax.ShapeDtypeStruct((_NW, _L), jnp.float32),
      mesh=plsc.VectorSubcoreMesh(core_axis_name="c", subcore_axis_name="s",
                                  num_cores=_NC, num_subcores=_NS),
      compiler_params=pltpu.CompilerParams(needs_layout_passes=False),
      scratch_types=[
          pltpu.VMEM((_WIN,), jnp.float32),
          pltpu.VMEM((_WIN,), jnp.float32),
          pltpu.VMEM((_L,), jnp.float32),
          pltpu.VMEM((_L,), jnp.float32),
          pltpu.SemaphoreType.DMA,
          pltpu.SemaphoreType.DMA,
      ],
  )


def _atan(u):
  t = jnp.abs(u)
  inv = t > 1.0
  z = jnp.where(inv, 1.0 / jnp.maximum(t, 1e-30), t)
  big = z > 0.4142135623730951
  z2 = jnp.where(big, (z - 1.0) / (z + 1.0), z)
  w = z2 * z2
  p = ((8.05374449538e-2 * w - 1.38776856032e-1) * w
       + 1.99777106478e-1) * w - 3.33329491539e-1
  r = z2 + z2 * w * p
  r = jnp.where(big, jnp.float32(0.7853981633974483) + r, r)
  r = jnp.where(inv, jnp.float32(1.5707963267948966) - r, r)
  return jnp.where(u < 0.0, -r, r)


def _merge_body(pv_ref, cand_ref, wt_ref, out_ref, row_v, sem):
  cand = cand_ref[...]
  lane = lax.broadcasted_iota(jnp.int32, (_NW, _L), 1)
  wrow = lax.broadcasted_iota(jnp.int32, (_NW, 1), 0).astype(jnp.float32)
  d2f = jnp.where(lane == 0, cand, _BIG)
  m = jnp.min(d2f)
  rowd2 = jnp.min(d2f, axis=1, keepdims=True)
  rowidx = jnp.max(jnp.where(lane == 1, cand, -_BIG), axis=1, keepdims=True)
  key = rowidx * jnp.float32(_NW) + wrow
  keysel = jnp.where(rowd2 <= m, key, _BIG)
  kstar = jnp.min(keysel)
  istar = jnp.floor(kstar / jnp.float32(_NW)).astype(jnp.int32)

  c0 = pl.multiple_of((istar // 128) * 128, 128)
  cp = pltpu.make_async_copy(wt_ref.at[:, pl.ds(c0, 128)], row_v, sem)
  cp.start()
  cp.wait()
  j = istar - c0
  li = lax.broadcasted_iota(jnp.int32, (6, 128), 1)
  ri = lax.broadcasted_iota(jnp.int32, (6, 128), 0)
  rv = row_v[...]
  selc = li == j
  wx = jnp.sum(jnp.where(selc & (ri == 1), rv, 0.0))
  wy = jnp.sum(jnp.where(selc & (ri == 2), rv, 0.0))
  wh = jnp.sum(jnp.where(selc & (ri == 3), rv, 0.0))
  ws = jnp.sum(jnp.where(selc & (ri == 5), rv, 0.0))

  pcx = pv_ref[0]
  pcy = pv_ref[1]
  fav0 = pv_ref[2]
  fav1 = pv_ref[3]
  thetap = pv_ref[4]
  k_e = pv_ref[5]
  k_h = pv_ref[6]

  ce = (pcx - wx) * fav0 + (pcy - wy) * fav1
  pi = jnp.float32(jnp.pi)
  he = jnp.remainder(wh - thetap + pi, 2.0 * pi) - pi
  v = ws * jnp.float32(_VGOAL)
  steer = k_h * he + _atan(k_e * -ce / (v + 1e-05))
  ol = lax.broadcasted_iota(jnp.int32, (1, 8), 1)
  res = jnp.where(ol == 0, steer,
                  jnp.where(ol == 1, v,
                            jnp.where(ol == 2, ce, he)))
  out_ref[...] = res


_merge = pl.pallas_call(
    _merge_body,
    out_shape=jax.ShapeDtypeStruct((1, 8), jnp.float32),
    in_specs=[
        pl.BlockSpec(memory_space=pltpu.SMEM),
        pl.BlockSpec(memory_space=pltpu.VMEM),
        pl.BlockSpec(memory_space=pl.ANY),
    ],
    out_specs=pl.BlockSpec(memory_space=pltpu.VMEM),
    scratch_shapes=[
        pltpu.VMEM((6, 128), jnp.float32),
        pltpu.SemaphoreType.DMA,
    ],
)


def kernel(pose, waypoints, k_e, k_h):
  pcx = pose[0] + _LF * jnp.sin(pose[2])
  pcy = pose[1] + _LF * jnp.cos(pose[2])
  thetap = jnp.remainder(pose[2] + jnp.pi, 2.0 * jnp.pi)
  fav0 = -jnp.cos(thetap + jnp.pi / 2.0)
  fav1 = -jnp.sin(thetap + jnp.pi / 2.0)
  pvec_sc = jnp.zeros((_L,), jnp.float32).at[0].set(pcx).at[1].set(pcy)
  wx = waypoints[:, 1]
  wy = waypoints[:, 2]
  wt = waypoints.T
  cand = _get_sc_search()(wx, wy, pvec_sc)
  pvec_tc = jnp.stack([pcx, pcy, fav0, fav1, thetap,
                       k_e.astype(jnp.float32), k_h.astype(jnp.float32),
                       jnp.float32(0.0)])
  out = _merge(pvec_tc, cand, wt)
  return (out[0, 0], out[0, 1], out[0, 2], out[0, 3])
